# baseline (speedup 1.0000x reference)
import functools

import jax
import jax.numpy as jnp
from jax import lax
from jax.experimental import pallas as pl
from jax.experimental.pallas import tpu as pltpu
from jax.experimental.pallas import tpu_sc as plsc

N = 10000
E = 320000
D = 128
EPS = 1e-5

NC = 2
NS = 16
NW = NC * NS
BLK = 128
EPW_BLKS = 80
SLAB = EPW_BLKS // 2
E_PAD = NW * EPW_BLKS * BLK
N_PAD = 10112
ROWS_PER_SUB = N_PAD // NS
NB = 2


def _sc_segment_sum(xg, src2d, dst2d):
    mesh = plsc.VectorSubcoreMesh(
        core_axis_name="c", subcore_axis_name="s", num_cores=NC,
        num_subcores=NS)

    @functools.partial(
        pl.kernel,
        out_type=jax.ShapeDtypeStruct((NC, N_PAD, D), jnp.float32),
        mesh=mesh,
        scratch_types=dict(
            acc=pltpu.VMEM_SHARED((N_PAD, D), jnp.float32),
            idx_s=pltpu.VMEM((SLAB, BLK), jnp.int32),
            idx_d=pltpu.VMEM((SLAB, BLK), jnp.int32),
            rows=pltpu.VMEM((NB, BLK, D), jnp.float32),
            sem_g=pltpu.SemaphoreType.DMA,
            sem_s=pltpu.SemaphoreType.DMA,
        ),
    )
    def k(xg_hbm, src_hbm, dst_hbm, out_hbm, acc, idx_s, idx_d, rows,
          sem_g, sem_s):
        c = lax.axis_index("c")
        s = lax.axis_index("s")
        w = c * NS + s

        def zrow(i, _):
            for j in range(D // 16):
                rows[0, i, pl.ds(j * 16, 16)] = jnp.zeros((16,), jnp.float32)
            return 0
        lax.fori_loop(0, BLK, zrow, 0)
        base = s * ROWS_PER_SUB
        off = 0
        for cnt in (128, 128, 128, 128, 120):
            pltpu.sync_copy(rows.at[0, pl.ds(0, cnt)],
                            acc.at[pl.ds(base + off, cnt)])
            off += cnt
        plsc.subcore_barrier()

        def scatter_wait():
            pltpu.make_async_copy(
                rows.at[0], acc.at[idx_d.at[0]], sem_s).wait()

        def fire(r0, first):
            gds = []
            for b in range(NB):
                if not first:
                    scatter_wait()
                gds.append(pltpu.async_copy(
                    xg_hbm.at[idx_s.at[r0 + b]], rows.at[b], sem_g))
            for b in range(NB):
                gds[b].wait()
                pltpu.make_async_copy(
                    rows.at[b], acc.at[idx_d.at[r0 + b]],
                    sem_s).start(add=True)

        for h in range(2):
            if h:
                for _ in range(NB):
                    scatter_wait()
            slab0 = w * EPW_BLKS + h * SLAB
            pltpu.sync_copy(src_hbm.at[pl.ds(slab0, SLAB)], idx_s)
            pltpu.sync_copy(dst_hbm.at[pl.ds(slab0, SLAB)], idx_d)

            fire(0, first=True)

            def body(g, _):
                fire(g * NB, first=False)
                return 0
            lax.fori_loop(1, SLAB // NB, body, 0)
        for _ in range(NB):
            scatter_wait()
        plsc.subcore_barrier()

        off = 0
        for cnt in (128, 128, 128, 128, 120):
            pltpu.sync_copy(acc.at[pl.ds(base + off, cnt)],
                            out_hbm.at[c, pl.ds(base + off, cnt)])
            off += cnt

    return k(xg, src2d, dst2d)


def _tc_dense(p, x, W_rel, b_rel, W_root, gamma, beta):
    def body(p_ref, x_ref, wrel_ref, brel_ref, wroot_ref, g_ref, b_ref,
             o_ref):
        agg = p_ref[0, :N] + p_ref[1, :N]
        out = (
            jnp.dot(agg, wrel_ref[...].T, preferred_element_type=jnp.float32)
            + brel_ref[...][None, :]
            + jnp.dot(x_ref[...], wroot_ref[...].T,
                      preferred_element_type=jnp.float32)
        )
        mean = jnp.mean(out, axis=0)
        cen = out - mean[None, :]
        var = jnp.mean(cen * cen, axis=0)
        h = cen * lax.rsqrt(var + EPS) * g_ref[...][None, :] + b_ref[...][None, :]
        o_ref[...] = jnp.maximum(h, 0.0)

    return pl.pallas_call(
        body,
        out_shape=jax.ShapeDtypeStruct((N, D), jnp.float32),
    )(p, x, W_rel, b_rel, W_root, gamma, beta)


def kernel(x, edge_index, W_rel, b_rel, W_root, gamma, beta):
    n_pad = E_PAD - E
    i = lax.iota(jnp.int32, n_pad)
    src = jnp.concatenate([edge_index[0], i % BLK])
    dst = jnp.concatenate([edge_index[1], N + (i % (N_PAD - N))])
    src2d = src.reshape(E_PAD // BLK, BLK)
    dst2d = dst.reshape(E_PAD // BLK, BLK)

    p = _sc_segment_sum(x, src2d, dst2d)
    return _tc_dense(p, x, W_rel, b_rel, W_root, gamma, beta)


# speedup vs baseline: 1.0510x; 1.0510x over previous
import functools

import jax
import jax.numpy as jnp
from jax import lax
from jax.experimental import pallas as pl
from jax.experimental.pallas import tpu as pltpu
from jax.experimental.pallas import tpu_sc as plsc

N = 10000
E = 320000
D = 128
EPS = 1e-5

NC = 2
NS = 16
NW = NC * NS
BLK = 128
EPW_BLKS = 81
E_PAD = NW * EPW_BLKS * BLK
N_PAD = 10112
ROWS_PER_SUB = N_PAD // NS
NB = 3


def _sc_segment_sum(xg, src2d, dst2d):
    mesh = plsc.VectorSubcoreMesh(
        core_axis_name="c", subcore_axis_name="s", num_cores=NC,
        num_subcores=NS)

    @functools.partial(
        pl.kernel,
        out_type=jax.ShapeDtypeStruct((NC, N_PAD, D), jnp.float32),
        mesh=mesh,
        scratch_types=dict(
            acc=pltpu.VMEM_SHARED((N_PAD, D), jnp.float32),
            idx_s=pltpu.VMEM((NB, BLK), jnp.int32),
            idx_d=pltpu.VMEM((NB, BLK), jnp.int32),
            rows=pltpu.VMEM((NB, BLK, D), jnp.float32),
            sem_i=pltpu.SemaphoreType.DMA,
            sem_g=pltpu.SemaphoreType.DMA,
            sem_s=pltpu.SemaphoreType.DMA,
        ),
    )
    def k(xg_hbm, src_hbm, dst_hbm, out_hbm, acc, idx_s, idx_d, rows,
          sem_i, sem_g, sem_s):
        c = lax.axis_index("c")
        s = lax.axis_index("s")
        w = c * NS + s

        def zrow(i, _):
            for j in range(D // 16):
                rows[0, i, pl.ds(j * 16, 16)] = jnp.zeros((16,), jnp.float32)
            return 0
        lax.fori_loop(0, BLK, zrow, 0)
        base = s * ROWS_PER_SUB
        off = 0
        for cnt in (128, 128, 128, 128, 120):
            pltpu.sync_copy(rows.at[0, pl.ds(0, cnt)],
                            acc.at[pl.ds(base + off, cnt)])
            off += cnt
        plsc.subcore_barrier()

        row0 = w * EPW_BLKS

        def wait_i():
            pltpu.make_async_copy(
                src_hbm.at[row0], idx_s.at[0], sem_i).wait()

        def wait_g():
            pltpu.make_async_copy(
                xg_hbm.at[idx_s.at[0]], rows.at[0], sem_g).wait()

        def wait_s():
            pltpu.make_async_copy(
                rows.at[0], acc.at[idx_d.at[0]], sem_s).wait()

        def stage_a(r, slot, first=False):
            if not first:
                wait_s()
            pltpu.async_copy(src_hbm.at[row0 + r], idx_s.at[slot], sem_i)
            pltpu.async_copy(dst_hbm.at[row0 + r], idx_d.at[slot], sem_i)

        def stage_b(slot):
            wait_i()
            wait_i()
            pltpu.async_copy(
                xg_hbm.at[idx_s.at[slot]], rows.at[slot], sem_g)

        def stage_c(slot):
            wait_g()
            pltpu.make_async_copy(
                rows.at[slot], acc.at[idx_d.at[slot]],
                sem_s).start(add=True)

        stage_a(0, 0, first=True)
        stage_b(0)
        stage_a(1, 1, first=True)
        stage_b(1)
        stage_c(0)
        stage_a(2, 2, first=True)

        def body(g, _):
            r0 = 3 * g
            for b in range(3):
                stage_b((b + 2) % 3)
                stage_c((b + 1) % 3)
                stage_a(r0 + b, b)
            return 0
        lax.fori_loop(1, EPW_BLKS // 3, body, 0)

        stage_b(2)
        stage_c(1)
        stage_c(2)
        for _ in range(3):
            wait_s()
        plsc.subcore_barrier()

        off = 0
        for cnt in (128, 128, 128, 128, 120):
            pltpu.sync_copy(acc.at[pl.ds(base + off, cnt)],
                            out_hbm.at[c, pl.ds(base + off, cnt)])
            off += cnt

    return k(xg, src2d, dst2d)


def _tc_dense(p, x, W_rel, b_rel, W_root, gamma, beta):
    def body(p_ref, x_ref, wrel_ref, brel_ref, wroot_ref, g_ref, b_ref,
             o_ref):
        agg = p_ref[0, :N] + p_ref[1, :N]
        out = (
            jnp.dot(agg, wrel_ref[...].T, preferred_element_type=jnp.float32)
            + brel_ref[...][None, :]
            + jnp.dot(x_ref[...], wroot_ref[...].T,
                      preferred_element_type=jnp.float32)
        )
        mean = jnp.mean(out, axis=0)
        cen = out - mean[None, :]
        var = jnp.mean(cen * cen, axis=0)
        h = cen * lax.rsqrt(var + EPS) * g_ref[...][None, :] + b_ref[...][None, :]
        o_ref[...] = jnp.maximum(h, 0.0)

    return pl.pallas_call(
        body,
        out_shape=jax.ShapeDtypeStruct((N, D), jnp.float32),
    )(p, x, W_rel, b_rel, W_root, gamma, beta)


def kernel(x, edge_index, W_rel, b_rel, W_root, gamma, beta):
    n_pad = E_PAD - E
    i = lax.iota(jnp.int32, n_pad)
    src = jnp.concatenate([edge_index[0], i % BLK])
    dst = jnp.concatenate([edge_index[1], N + (i % (N_PAD - N))])
    src2d = src.reshape(E_PAD // BLK, BLK)
    dst2d = dst.reshape(E_PAD // BLK, BLK)

    p = _sc_segment_sum(x, src2d, dst2d)
    return _tc_dense(p, x, W_rel, b_rel, W_root, gamma, beta)


# speedup vs baseline: 1.1232x; 1.0687x over previous
import functools

import jax
import jax.numpy as jnp
from jax import lax
from jax.experimental import pallas as pl
from jax.experimental.pallas import tpu as pltpu
from jax.experimental.pallas import tpu_sc as plsc

N = 10000
E = 320000
D = 128
EPS = 1e-5

NC = 2
NS = 16
NW = NC * NS
BLK = 128
EPW_BLKS = 81
E_PAD = NW * EPW_BLKS * BLK
N_PAD = 10112
ROWS_PER_SUB = N_PAD // NS
NB = 3


NBLK_REAL = E // BLK
NBLK_PAD = E_PAD // BLK - NBLK_REAL


def _sc_segment_sum(xg, e3, pad_s, pad_d):
    mesh = plsc.VectorSubcoreMesh(
        core_axis_name="c", subcore_axis_name="s", num_cores=NC,
        num_subcores=NS)

    @functools.partial(
        pl.kernel,
        out_type=jax.ShapeDtypeStruct((NC, N_PAD, D), jnp.float32),
        mesh=mesh,
        scratch_types=dict(
            acc=pltpu.VMEM_SHARED((N_PAD, D), jnp.float32),
            idx_s=pltpu.VMEM((NB, BLK), jnp.int32),
            idx_d=pltpu.VMEM((NB, BLK), jnp.int32),
            rows=pltpu.VMEM((NB, BLK, D), jnp.float32),
            sem_i=pltpu.SemaphoreType.DMA,
            sem_g=pltpu.SemaphoreType.DMA,
            sem_s=pltpu.SemaphoreType.DMA,
        ),
    )
    def k(xg_hbm, e_hbm, pads_hbm, padd_hbm, out_hbm, acc, idx_s, idx_d,
          rows, sem_i, sem_g, sem_s):
        c = lax.axis_index("c")
        s = lax.axis_index("s")
        w = c * NS + s

        def zrow(i, _):
            for j in range(D // 16):
                rows[0, i, pl.ds(j * 16, 16)] = jnp.zeros((16,), jnp.float32)
            return 0
        lax.fori_loop(0, BLK, zrow, 0)
        base = s * ROWS_PER_SUB
        off = 0
        for cnt in (128, 128, 128, 128, 120):
            pltpu.sync_copy(rows.at[0, pl.ds(0, cnt)],
                            acc.at[pl.ds(base + off, cnt)])
            off += cnt
        plsc.subcore_barrier()

        row0 = w * EPW_BLKS

        def wait_i():
            pltpu.make_async_copy(
                pads_hbm.at[0], idx_s.at[0], sem_i).wait()

        def wait_g():
            pltpu.make_async_copy(
                xg_hbm.at[idx_s.at[0]], rows.at[0], sem_g).wait()

        def wait_s():
            pltpu.make_async_copy(
                rows.at[0], acc.at[idx_d.at[0]], sem_s).wait()

        def stage_a(r, slot, first=False):
            if not first:
                wait_s()
            row = row0 + r

            def real(_):
                pltpu.async_copy(e_hbm.at[0, row], idx_s.at[slot], sem_i)
                pltpu.async_copy(e_hbm.at[1, row], idx_d.at[slot], sem_i)
                return 0

            def pad(_):
                prow = row - NBLK_REAL
                pltpu.async_copy(pads_hbm.at[prow], idx_s.at[slot], sem_i)
                pltpu.async_copy(padd_hbm.at[prow], idx_d.at[slot], sem_i)
                return 0

            lax.cond(row < NBLK_REAL, real, pad, 0)

        def stage_b(slot):
            wait_i()
            wait_i()
            pltpu.async_copy(
                xg_hbm.at[idx_s.at[slot]], rows.at[slot], sem_g)

        def stage_c(slot):
            wait_g()
            pltpu.make_async_copy(
                rows.at[slot], acc.at[idx_d.at[slot]],
                sem_s).start(add=True)

        stage_a(0, 0, first=True)
        stage_b(0)
        stage_a(1, 1, first=True)
        stage_b(1)
        stage_c(0)
        stage_a(2, 2, first=True)

        def body(g, _):
            r0 = 3 * g
            for b in range(3):
                stage_b((b + 2) % 3)
                stage_c((b + 1) % 3)
                stage_a(r0 + b, b)
            return 0
        lax.fori_loop(1, EPW_BLKS // 3, body, 0)

        stage_b(2)
        stage_c(1)
        stage_c(2)
        for _ in range(3):
            wait_s()
        plsc.subcore_barrier()

        off = 0
        for cnt in (128, 128, 128, 128, 120):
            pltpu.sync_copy(acc.at[pl.ds(base + off, cnt)],
                            out_hbm.at[c, pl.ds(base + off, cnt)])
            off += cnt

    return k(xg, e3, pad_s, pad_d)


def _tc_dense(p, x, W_rel, b_rel, W_root, gamma, beta):
    def body(p_ref, x_ref, wrel_ref, brel_ref, wroot_ref, g_ref, b_ref,
             o_ref):
        agg = p_ref[0, :N] + p_ref[1, :N]
        out = (
            jnp.dot(agg, wrel_ref[...].T, preferred_element_type=jnp.float32)
            + brel_ref[...][None, :]
            + jnp.dot(x_ref[...], wroot_ref[...].T,
                      preferred_element_type=jnp.float32)
        )
        mean = jnp.mean(out, axis=0)
        cen = out - mean[None, :]
        var = jnp.mean(cen * cen, axis=0)
        h = cen * lax.rsqrt(var + EPS) * g_ref[...][None, :] + b_ref[...][None, :]
        o_ref[...] = jnp.maximum(h, 0.0)

    return pl.pallas_call(
        body,
        out_shape=jax.ShapeDtypeStruct((N, D), jnp.float32),
    )(p, x, W_rel, b_rel, W_root, gamma, beta)


def kernel(x, edge_index, W_rel, b_rel, W_root, gamma, beta):
    e3 = edge_index.reshape(2, NBLK_REAL, BLK)
    i = lax.iota(jnp.int32, NBLK_PAD * BLK)
    pad_s = (i % BLK).reshape(NBLK_PAD, BLK)
    pad_d = (N + (i % (N_PAD - N))).reshape(NBLK_PAD, BLK)

    p = _sc_segment_sum(x, e3, pad_s, pad_d)
    return _tc_dense(p, x, W_rel, b_rel, W_root, gamma, beta)
